# baseline (speedup 1.0000x reference)
import functools

import jax
import jax.numpy as jnp
from jax import lax
from jax.experimental import pallas as pl
from jax.experimental.pallas import tpu as pltpu


def _round_up(x, m):
    return ((x + m - 1) // m) * m


def _fwd_kernel(x_ref, w_ref, b_ref, c_ref, emb_ref, out_ref,
                wb_ref, cnb_ref):
    i = pl.program_id(0)

    @pl.when(i == 0)
    def _init():
        wb_ref[...] = w_ref[...].astype(jnp.bfloat16)
        c = c_ref[...]
        inv_c = lax.rsqrt(jnp.sum(c * c, axis=-1, keepdims=True))
        cnb_ref[...] = (c * inv_c).astype(jnp.bfloat16)

    xb = x_ref[...].astype(jnp.bfloat16)
    emb = jnp.dot(xb, wb_ref[...],
                  preferred_element_type=jnp.float32) + b_ref[...]
    emb_ref[...] = emb

    inv = lax.rsqrt(jnp.sum(emb * emb, axis=-1, keepdims=True))
    emb_n = (emb * inv).astype(jnp.bfloat16)

    out_ref[...] = lax.dot_general(
        emb_n, cnb_ref[...],
        dimension_numbers=(((1,), (1,)), ((), ())),
        preferred_element_type=jnp.float32)


@functools.partial(jax.jit, static_argnames=("block_b",))
def _forward(x, w, b, centroids, *, block_b=1024):
    B, D = x.shape
    Dw, H = w.shape
    C, Hc = centroids.shape
    assert D == Dw and H == Hc

    f32 = jnp.float32
    bf16 = jnp.bfloat16

    D_p = _round_up(D, 128)
    H_p = _round_up(H, 128)
    C_p = _round_up(C, 128)

    if B < block_b:
        bb = _round_up(B, 8)
        B_p = bb
    else:
        bb = block_b
        B_p = _round_up(B, bb)
    n_blk = B_p // bb

    if (B_p, D_p) == (B, D):
        x_p = x
    else:
        x_p = jnp.zeros((B_p, D_p), f32).at[:B, :D].set(x)
    if (D_p, H_p) == (D, H):
        w_p = w
    else:
        w_p = jnp.zeros((D_p, H_p), f32).at[:D, :H].set(w)
    if H_p == H:
        b_p = b.reshape(1, H)
    else:
        b_p = jnp.zeros((1, H_p), f32).at[:, :H].set(b.reshape(1, H))
    if (C_p, H_p) == (C, H):
        c_p = centroids
    else:
        c_p = jnp.zeros((C_p, H_p), f32).at[:C, :H].set(centroids)
        c_p = c_p.at[C:, 0].set(1.0)

    emb_p, out_p = pl.pallas_call(
        _fwd_kernel,
        out_shape=(
            jax.ShapeDtypeStruct((B_p, H_p), f32),
            jax.ShapeDtypeStruct((B_p, C_p), f32),
        ),
        grid=(n_blk,),
        in_specs=[
            pl.BlockSpec((bb, D_p), lambda i: (i, 0)),
            pl.BlockSpec((D_p, H_p), lambda i: (0, 0)),
            pl.BlockSpec((1, H_p), lambda i: (0, 0)),
            pl.BlockSpec((C_p, H_p), lambda i: (0, 0)),
        ],
        out_specs=(
            pl.BlockSpec((bb, H_p), lambda i: (i, 0)),
            pl.BlockSpec((bb, C_p), lambda i: (i, 0)),
        ),
        scratch_shapes=[
            pltpu.VMEM((D_p, H_p), bf16),
            pltpu.VMEM((C_p, H_p), bf16),
        ],
        compiler_params=pltpu.CompilerParams(
            dimension_semantics=("arbitrary",),
            vmem_limit_bytes=100 * 1024 * 1024,
        ),
    )(x_p, w_p, b_p, c_p)

    if (B_p, H_p, C_p) == (B, H, C):
        return emb_p, out_p
    return emb_p[:B, :H], out_p[:B, :C]


def kernel(x, w, b, centroids):
    return _forward(x, w, b, centroids, block_b=512)


# speedup vs baseline: 1.1238x; 1.1238x over previous
import functools

import jax
import jax.numpy as jnp
from jax import lax
from jax.experimental import pallas as pl
from jax.experimental.pallas import tpu as pltpu


def _round_up(x, m):
    return ((x + m - 1) // m) * m


def _fwd_kernel(x_ref, w_ref, b_ref, c_ref, emb_ref, out_ref,
                wb_ref, cnb_ref):
    i = pl.program_id(0)

    @pl.when(i == 0)
    def _init():
        wb_ref[...] = w_ref[...].astype(jnp.bfloat16)
        c = c_ref[...]
        inv_c = lax.rsqrt(jnp.sum(c * c, axis=-1, keepdims=True))
        cnb_ref[...] = (c * inv_c).astype(jnp.bfloat16)

    xb = x_ref[...].astype(jnp.bfloat16)
    emb = jnp.dot(xb, wb_ref[...],
                  preferred_element_type=jnp.float32) + b_ref[...]
    emb_ref[...] = emb

    inv = lax.rsqrt(jnp.sum(emb * emb, axis=-1, keepdims=True))
    emb_n = (emb * inv).astype(jnp.bfloat16)

    out_ref[...] = lax.dot_general(
        emb_n, cnb_ref[...],
        dimension_numbers=(((1,), (1,)), ((), ())),
        preferred_element_type=jnp.float32)


@functools.partial(jax.jit, static_argnames=("block_b",))
def _forward(x, w, b, centroids, *, block_b=1024):
    B, D = x.shape
    Dw, H = w.shape
    C, Hc = centroids.shape
    assert D == Dw and H == Hc

    f32 = jnp.float32
    bf16 = jnp.bfloat16

    D_p = _round_up(D, 128)
    H_p = _round_up(H, 128)
    C_p = _round_up(C, 128)

    if B < block_b:
        bb = _round_up(B, 8)
        B_p = bb
    else:
        bb = block_b
        B_p = _round_up(B, bb)
    n_blk = B_p // bb

    if (B_p, D_p) == (B, D):
        x_p = x
    else:
        x_p = jnp.zeros((B_p, D_p), f32).at[:B, :D].set(x)
    if (D_p, H_p) == (D, H):
        w_p = w
    else:
        w_p = jnp.zeros((D_p, H_p), f32).at[:D, :H].set(w)
    if H_p == H:
        b_p = b.reshape(1, H)
    else:
        b_p = jnp.zeros((1, H_p), f32).at[:, :H].set(b.reshape(1, H))
    if (C_p, H_p) == (C, H):
        c_p = centroids
    else:
        c_p = jnp.zeros((C_p, H_p), f32).at[:C, :H].set(centroids)
        c_p = c_p.at[C:, 0].set(1.0)

    emb_p, out_p = pl.pallas_call(
        _fwd_kernel,
        out_shape=(
            jax.ShapeDtypeStruct((B_p, H_p), f32),
            jax.ShapeDtypeStruct((B_p, C_p), f32),
        ),
        grid=(n_blk,),
        in_specs=[
            pl.BlockSpec((bb, D_p), lambda i: (i, 0)),
            pl.BlockSpec((D_p, H_p), lambda i: (0, 0)),
            pl.BlockSpec((1, H_p), lambda i: (0, 0)),
            pl.BlockSpec((C_p, H_p), lambda i: (0, 0)),
        ],
        out_specs=(
            pl.BlockSpec((bb, H_p), lambda i: (i, 0)),
            pl.BlockSpec((bb, C_p), lambda i: (i, 0)),
        ),
        scratch_shapes=[
            pltpu.VMEM((D_p, H_p), bf16),
            pltpu.VMEM((C_p, H_p), bf16),
        ],
        compiler_params=pltpu.CompilerParams(
            dimension_semantics=("arbitrary",),
            vmem_limit_bytes=120 * 1024 * 1024,
        ),
    )(x_p, w_p, b_p, c_p)

    if (B_p, H_p, C_p) == (B, H, C):
        return emb_p, out_p
    return emb_p[:B, :H], out_p[:B, :C]


def kernel(x, w, b, centroids):
    return _forward(x, w, b, centroids, block_b=1024)


# speedup vs baseline: 1.1322x; 1.0075x over previous
import functools

import jax
import jax.numpy as jnp
from jax import lax
from jax.experimental import pallas as pl
from jax.experimental.pallas import tpu as pltpu


def _round_up(x, m):
    return ((x + m - 1) // m) * m


def _fwd_kernel(x_ref, w_ref, b_ref, c_ref, emb_ref, out_ref,
                wb_ref, cnb_ref):
    i = pl.program_id(0)

    @pl.when(i == 0)
    def _init():
        wb_ref[...] = w_ref[...].astype(jnp.bfloat16)
        c = c_ref[...]
        inv_c = lax.rsqrt(jnp.sum(c * c, axis=-1, keepdims=True))
        cnb_ref[...] = (c * inv_c).astype(jnp.bfloat16)

    xb = x_ref[...].astype(jnp.bfloat16)
    emb = jnp.dot(xb, wb_ref[...],
                  preferred_element_type=jnp.float32) + b_ref[...]
    emb_ref[...] = emb

    raw = lax.dot_general(
        emb.astype(jnp.bfloat16), cnb_ref[...],
        dimension_numbers=(((1,), (1,)), ((), ())),
        preferred_element_type=jnp.float32)
    inv = lax.rsqrt(jnp.sum(emb * emb, axis=-1, keepdims=True))
    out_ref[...] = raw * inv


@functools.partial(jax.jit, static_argnames=("block_b",))
def _forward(x, w, b, centroids, *, block_b=1024):
    B, D = x.shape
    Dw, H = w.shape
    C, Hc = centroids.shape
    assert D == Dw and H == Hc

    f32 = jnp.float32
    bf16 = jnp.bfloat16

    D_p = _round_up(D, 128)
    H_p = _round_up(H, 128)
    C_p = _round_up(C, 128)

    if B < block_b:
        bb = _round_up(B, 8)
        B_p = bb
    else:
        bb = block_b
        B_p = _round_up(B, bb)
    n_blk = B_p // bb

    if (B_p, D_p) == (B, D):
        x_p = x
    else:
        x_p = jnp.zeros((B_p, D_p), f32).at[:B, :D].set(x)
    if (D_p, H_p) == (D, H):
        w_p = w
    else:
        w_p = jnp.zeros((D_p, H_p), f32).at[:D, :H].set(w)
    if H_p == H:
        b_p = b.reshape(1, H)
    else:
        b_p = jnp.zeros((1, H_p), f32).at[:, :H].set(b.reshape(1, H))
    if (C_p, H_p) == (C, H):
        c_p = centroids
    else:
        c_p = jnp.zeros((C_p, H_p), f32).at[:C, :H].set(centroids)
        c_p = c_p.at[C:, 0].set(1.0)

    emb_p, out_p = pl.pallas_call(
        _fwd_kernel,
        out_shape=(
            jax.ShapeDtypeStruct((B_p, H_p), f32),
            jax.ShapeDtypeStruct((B_p, C_p), f32),
        ),
        grid=(n_blk,),
        in_specs=[
            pl.BlockSpec((bb, D_p), lambda i: (i, 0)),
            pl.BlockSpec((D_p, H_p), lambda i: (0, 0)),
            pl.BlockSpec((1, H_p), lambda i: (0, 0)),
            pl.BlockSpec((C_p, H_p), lambda i: (0, 0)),
        ],
        out_specs=(
            pl.BlockSpec((bb, H_p), lambda i: (i, 0)),
            pl.BlockSpec((bb, C_p), lambda i: (i, 0)),
        ),
        scratch_shapes=[
            pltpu.VMEM((D_p, H_p), bf16),
            pltpu.VMEM((C_p, H_p), bf16),
        ],
        compiler_params=pltpu.CompilerParams(
            dimension_semantics=("arbitrary",),
            vmem_limit_bytes=120 * 1024 * 1024,
        ),
    )(x_p, w_p, b_p, c_p)

    if (B_p, H_p, C_p) == (B, H, C):
        return emb_p, out_p
    return emb_p[:B, :H], out_p[:B, :C]


def kernel(x, w, b, centroids):
    return _forward(x, w, b, centroids, block_b=1024)
